# baseline (device time: 70160 ns/iter reference)
import os

import jax
import jax.numpy as jnp
from jax import lax
from jax.experimental import pallas as pl
from jax.experimental.pallas import tpu as pltpu

N_DEV = 16
G = int(os.environ.get("A2A_G", "4"))
KC = int(os.environ.get("A2A_KC", "2048"))

_SKIP_COMM = os.environ.get("SKIP_COMM") == "1"


def kernel(x, w_mat):
    m_per, k = x.shape
    _, n = w_mat.shape
    n_per = n // N_DEV
    blk = n_per * G
    n_grp = N_DEV // G
    n_kc = k // KC

    def body(x_ref, w_hbm, out_ref, w_buf, y_buf, w_sems, send_sems, recv_sem):
        my_i = lax.axis_index("i")
        my_grp = lax.div(my_i, G)

        def w_dma(slot, grp, kc):
            return pltpu.make_async_copy(
                w_hbm.at[pl.ds(kc * KC, KC), pl.ds(grp * blk, blk)],
                w_buf.at[slot],
                w_sems.at[slot],
            )

        def rdma(slot, d):
            return pltpu.make_async_remote_copy(
                src_ref=y_buf.at[slot],
                dst_ref=out_ref.at[pl.ds(my_i * m_per, m_per), :],
                send_sem=send_sems.at[slot],
                recv_sem=recv_sem,
                device_id=(d,),
                device_id_type=pl.DeviceIdType.MESH,
            )

        grp_of = lambda s: lax.rem(my_grp + s, n_grp)
        seq = [(s, kc) for s in range(n_grp) for kc in range(n_kc)]

        w_dma(0, grp_of(0), 0).start()

        for s in range(n_grp):
            grp = grp_of(s)
            acc = jnp.zeros((m_per, blk), jnp.float32)
            for kc in range(n_kc):
                t = s * n_kc + kc
                slot = t % 2
                if t + 1 < len(seq):
                    s2, kc2 = seq[t + 1]
                    w_dma(1 - slot, grp_of(s2), kc2).start()
                w_dma(slot, grp, kc).wait()
                acc = acc + jnp.dot(x_ref[:, kc * KC:(kc + 1) * KC],
                                    w_buf[slot],
                                    preferred_element_type=jnp.float32)
            for g in range(G):
                d = grp * G + g
                yslot = s * G + g
                y_blk = acc[:, g * n_per:(g + 1) * n_per]

                @pl.when(d == my_i)
                def _(y_blk=y_blk):
                    out_ref[pl.ds(my_i * m_per, m_per), :] = y_blk

                if not _SKIP_COMM:
                    @pl.when(d != my_i)
                    def _(yslot=yslot, d=d, y_blk=y_blk):
                        y_buf[yslot] = y_blk
                        rdma(yslot, d).start()

        if _SKIP_COMM:
            return

        for s in range(n_grp):
            grp = grp_of(s)
            for g in range(G):
                d = grp * G + g
                yslot = s * G + g

                @pl.when(d != my_i)
                def _(yslot=yslot, d=d):
                    rdma(yslot, d).wait_send()

        for _ in range(N_DEV - 1):
            recv = pltpu.make_async_remote_copy(
                src_ref=y_buf.at[0],
                dst_ref=out_ref.at[pl.ds(0, m_per), :],
                send_sem=send_sems.at[0],
                recv_sem=recv_sem,
                device_id=(my_i,),
                device_id_type=pl.DeviceIdType.MESH,
            )
            recv.wait_recv()

    return pl.pallas_call(
        body,
        out_shape=jax.ShapeDtypeStruct((N_DEV * m_per, n_per), jnp.float32),
        in_specs=[
            pl.BlockSpec(memory_space=pltpu.VMEM),
            pl.BlockSpec(memory_space=pl.ANY),
        ],
        out_specs=pl.BlockSpec(memory_space=pltpu.VMEM),
        scratch_shapes=[
            pltpu.VMEM((2, KC, blk), jnp.float32),
            pltpu.VMEM((N_DEV, m_per, n_per), jnp.float32),
            pltpu.SemaphoreType.DMA((2,)),
            pltpu.SemaphoreType.DMA((N_DEV,)),
            pltpu.SemaphoreType.DMA,
        ],
        compiler_params=pltpu.CompilerParams(
            vmem_limit_bytes=100 * 1024 * 1024,
        ),
    )(x, w_mat)
